# baseline (speedup 1.0000x reference)
import functools
import jax
import jax.numpy as jnp
from jax import lax
from jax.experimental import pallas as pl
from jax.experimental.pallas import tpu as pltpu
from jax.experimental.pallas import tpu_sc as plsc

_RS = 64
_D = 768
_LANES = 16


def _sc_add(in_flat, pos_table):
    n_rows, d = in_flat.shape
    seq = pos_table.shape[0]
    batch = n_rows // seq
    n_workers = 32
    seq_per_w = seq // n_workers
    steps = seq_per_w // _RS
    mesh = plsc.VectorSubcoreMesh(core_axis_name="c", subcore_axis_name="s")

    @functools.partial(
        pl.kernel,
        mesh=mesh,
        out_type=jax.ShapeDtypeStruct((n_rows, d), jnp.float32),
        scratch_types=[
            pltpu.VMEM((_RS, _D), jnp.float32),
            pltpu.VMEM((_RS, _D), jnp.float32),
        ],
    )
    def k(in_hbm, pos_hbm, out_hbm, pos_v, buf_v):
        wid = lax.axis_index("s") * 2 + lax.axis_index("c")
        seq0 = wid * seq_per_w

        def step(t, _):
            s0 = seq0 + t * _RS
            pltpu.sync_copy(pos_hbm.at[pl.ds(s0, _RS)], pos_v)

            def do_batch(b, _):
                row0 = b * seq + s0
                pltpu.sync_copy(in_hbm.at[pl.ds(row0, _RS)], buf_v)

                def add_row(r, _):
                    for c in range(_D // _LANES):
                        sl = pl.ds(c * _LANES, _LANES)
                        plsc.addupdate(buf_v.at[r, sl], pos_v[r, sl])
                    return ()

                lax.fori_loop(0, _RS, add_row, ())
                pltpu.sync_copy(buf_v, out_hbm.at[pl.ds(row0, _RS)])
                return ()

            lax.fori_loop(0, batch, do_batch, ())
            return ()

        lax.fori_loop(0, steps, step, ())

    return k(in_flat, pos_table)


def kernel(input_embeds, pos_table):
    b, s, d = input_embeds.shape
    out = _sc_add(input_embeds.reshape(b * s, d), pos_table)
    return out.reshape(b, s, d)


# speedup vs baseline: 2.1708x; 2.1708x over previous
import jax
import jax.numpy as jnp
from jax.experimental import pallas as pl


def _add_body(in_ref, pos_ref, out_ref):
    out_ref[...] = in_ref[...] + pos_ref[...]


def kernel(input_embeds, pos_table):
    batch, seq, d = input_embeds.shape
    S_BLK = 1024
    B_BLK = 1
    grid = (seq // S_BLK, batch // B_BLK)

    return pl.pallas_call(
        _add_body,
        grid=grid,
        in_specs=[
            pl.BlockSpec((B_BLK, S_BLK, d), lambda s, b: (b, s, 0)),
            pl.BlockSpec((S_BLK, d), lambda s, b: (s, 0)),
        ],
        out_specs=pl.BlockSpec((B_BLK, S_BLK, d), lambda s, b: (b, s, 0)),
        out_shape=jax.ShapeDtypeStruct((batch, seq, d), input_embeds.dtype),
    )(input_embeds, pos_table)


# speedup vs baseline: 2.3502x; 1.0826x over previous
import jax
import jax.numpy as jnp
from jax.experimental import pallas as pl
from jax.experimental.pallas import tpu as pltpu

_C = 2048
_NBUF = 2
_D = 768


def _body(in_hbm, pos_hbm, out_hbm, ibuf, obuf, posv, isems, osems, psems):
    n_rows = in_hbm.shape[0]
    seq = pos_hbm.shape[0]
    n_chunks = n_rows // _C
    pos_chunks = seq // _C

    def in_copy(c):
        return pltpu.make_async_copy(
            in_hbm.at[pl.ds(c * _C, _C)], ibuf.at[c % _NBUF], isems.at[c % _NBUF]
        )

    def out_copy(c):
        return pltpu.make_async_copy(
            obuf.at[c % _NBUF], out_hbm.at[pl.ds(c * _C, _C)], osems.at[c % _NBUF]
        )

    def pos_copy(p):
        return pltpu.make_async_copy(
            pos_hbm.at[pl.ds(p * _C, _C)], posv.at[pl.ds(p * _C, _C)], psems.at[p]
        )

    pos_copy(0).start()
    for k in range(_NBUF):
        in_copy(k).start()
    for p in range(1, pos_chunks):
        pos_copy(p).start()

    for c in range(n_chunks):
        slot = c % _NBUF
        if c >= _NBUF:
            out_copy(c - _NBUF).wait()
        in_copy(c).wait()
        if c < pos_chunks:
            pos_copy(c).wait()
        obuf[slot] = ibuf[slot] + posv[pl.ds((c * _C) % seq, _C)]
        out_copy(c).start()
        if c + _NBUF < n_chunks:
            in_copy(c + _NBUF).start()

    for c in range(n_chunks - _NBUF, n_chunks):
        out_copy(c).wait()


def kernel(input_embeds, pos_table):
    batch, seq, d = input_embeds.shape
    flat = input_embeds.reshape(batch * seq, d)

    out = pl.pallas_call(
        _body,
        in_specs=[
            pl.BlockSpec(memory_space=pl.ANY),
            pl.BlockSpec(memory_space=pl.ANY),
        ],
        out_specs=pl.BlockSpec(memory_space=pl.ANY),
        out_shape=jax.ShapeDtypeStruct((batch * seq, d), input_embeds.dtype),
        scratch_shapes=[
            pltpu.VMEM((_NBUF, _C, _D), jnp.float32),
            pltpu.VMEM((_NBUF, _C, _D), jnp.float32),
            pltpu.VMEM((8192, _D), jnp.float32),
            pltpu.SemaphoreType.DMA((_NBUF,)),
            pltpu.SemaphoreType.DMA((_NBUF,)),
            pltpu.SemaphoreType.DMA((8192 // _C,)),
        ],
    )(flat, pos_table)
    return out.reshape(batch, seq, d)


# speedup vs baseline: 2.3580x; 1.0033x over previous
import jax
import jax.numpy as jnp
from jax.experimental import pallas as pl
from jax.experimental.pallas import tpu as pltpu

_C = 1024
_NBUF = 4
_D = 768


def _body(in_hbm, pos_hbm, out_hbm, ibuf, obuf, posv, isems, osems, psems):
    n_rows = in_hbm.shape[0]
    seq = pos_hbm.shape[0]
    n_chunks = n_rows // _C
    pos_chunks = seq // _C

    def in_copy(c):
        return pltpu.make_async_copy(
            in_hbm.at[pl.ds(c * _C, _C)], ibuf.at[c % _NBUF], isems.at[c % _NBUF]
        )

    def out_copy(c):
        return pltpu.make_async_copy(
            obuf.at[c % _NBUF], out_hbm.at[pl.ds(c * _C, _C)], osems.at[c % _NBUF]
        )

    def pos_copy(p):
        return pltpu.make_async_copy(
            pos_hbm.at[pl.ds(p * _C, _C)], posv.at[pl.ds(p * _C, _C)], psems.at[p]
        )

    pos_copy(0).start()
    for k in range(_NBUF):
        in_copy(k).start()
        if k + 1 < pos_chunks:
            pos_copy(k + 1).start()
    for p in range(_NBUF + 1, pos_chunks):
        pos_copy(p).start()

    for c in range(n_chunks):
        slot = c % _NBUF
        if c >= _NBUF:
            out_copy(c - _NBUF).wait()
        in_copy(c).wait()
        if c < pos_chunks:
            pos_copy(c).wait()
        obuf[slot] = ibuf[slot] + posv[pl.ds((c * _C) % seq, _C)]
        out_copy(c).start()
        if c + _NBUF < n_chunks:
            in_copy(c + _NBUF).start()

    for c in range(n_chunks - _NBUF, n_chunks):
        out_copy(c).wait()


def kernel(input_embeds, pos_table):
    batch, seq, d = input_embeds.shape
    flat = input_embeds.reshape(batch * seq, d)

    out = pl.pallas_call(
        _body,
        in_specs=[
            pl.BlockSpec(memory_space=pl.ANY),
            pl.BlockSpec(memory_space=pl.ANY),
        ],
        out_specs=pl.BlockSpec(memory_space=pl.ANY),
        out_shape=jax.ShapeDtypeStruct((batch * seq, d), input_embeds.dtype),
        scratch_shapes=[
            pltpu.VMEM((_NBUF, _C, _D), jnp.float32),
            pltpu.VMEM((_NBUF, _C, _D), jnp.float32),
            pltpu.VMEM((8192, _D), jnp.float32),
            pltpu.SemaphoreType.DMA((_NBUF,)),
            pltpu.SemaphoreType.DMA((_NBUF,)),
            pltpu.SemaphoreType.DMA((8192 // _C,)),
        ],
    )(flat, pos_table)
    return out.reshape(batch, seq, d)
